# baseline (speedup 1.0000x reference)
import functools

import jax
import jax.numpy as jnp
from jax import lax
from jax.experimental import pallas as pl
from jax.experimental.pallas import tpu as pltpu
from jax.experimental.pallas import tpu_sc as plsc

VOCAB = 1000000
HID = 64
PAD = 0
SCALE = float(HID) ** 0.5

NC, NS, LANES = 2, 16, 16
NW = NC * NS
B_TOTAL = 4096 * 200
PER_W = B_TOTAL // NW
CHUNK = 200
NBUF = 2
N_CHUNKS = PER_W // CHUNK
assert N_CHUNKS % NBUF == 0


def _body(idx_hbm, table_hbm, out_hbm, idx_all,
          r0, r1, o0, o1, gs0, gs1, ss0, ss1):
  rbufs = [r0, r1]
  obufs = [o0, o1]
  gsems = [gs0, gs1]
  ssems = [ss0, ss1]

  wid = lax.axis_index("s") * NC + lax.axis_index("c")
  base = pl.multiple_of(wid * PER_W, 1024)

  pltpu.sync_copy(idx_hbm.at[pl.ds(base, PER_W)], idx_all)

  def start_gather(chunk, b):
    pltpu.async_copy(
        table_hbm.at[idx_all.at[pl.ds(chunk * CHUNK, CHUNK)]],
        rbufs[b], gsems[b])

  dnums = lax.GatherDimensionNumbers(
      offset_dims=(), collapsed_slice_dims=(0,), start_index_map=(0,))

  def splat(vec, j):
    return lax.gather(vec, jnp.full((LANES, 1), j, jnp.int32), dnums, (1,),
                      mode=lax.GatherScatterMode.PROMISE_IN_BOUNDS)

  def scale_chunk(chunk, b):
    rows = rbufs[b]
    dst = obufs[b]

    def group(g, carry):
      idxv = idx_all[pl.ds(chunk * CHUNK + g * LANES, LANES)]
      mv = jnp.minimum(idxv, 1).astype(jnp.float32) * SCALE
      for j in range(LANES):
        mspl = splat(mv, j)
        r = g * LANES + j
        for c in range(HID // LANES):
          dst[r, pl.ds(c * LANES, LANES)] = (
              rows[r, pl.ds(c * LANES, LANES)] * mspl)
      return carry

    lax.fori_loop(0, CHUNK // LANES, group, 0)

  for b in range(NBUF):
    start_gather(b, b)

  def outer(s, carry):
    for b in range(NBUF):
      chunk = s * NBUF + b
      pltpu.make_async_copy(
          table_hbm.at[idx_all.at[pl.ds(chunk * CHUNK, CHUNK)]],
          rbufs[b], gsems[b]).wait()

      @pl.when(chunk >= NBUF)
      def _():
        pltpu.make_async_copy(
            obufs[b], out_hbm.at[wid * N_CHUNKS + chunk - NBUF],
            ssems[b]).wait()

      scale_chunk(chunk, b)
      pltpu.async_copy(obufs[b], out_hbm.at[wid * N_CHUNKS + chunk], ssems[b])

      @pl.when(chunk + NBUF < N_CHUNKS)
      def _():
        start_gather(chunk + NBUF, b)

    return carry

  lax.fori_loop(0, N_CHUNKS // NBUF, outer, 0)

  for b in range(NBUF):
    chunk = N_CHUNKS - NBUF + b
    pltpu.make_async_copy(
        obufs[b], out_hbm.at[wid * N_CHUNKS + chunk], ssems[b]).wait()


@functools.partial(jax.jit, static_argnames=())
def _run(idx_flat, table):
  mesh = plsc.VectorSubcoreMesh(core_axis_name="c", subcore_axis_name="s")
  k = pl.kernel(
      _body,
      out_type=jax.ShapeDtypeStruct((4096, 200, HID), jnp.float32),
      mesh=mesh,
      scratch_types=(
          [pltpu.VMEM((PER_W,), jnp.int32)]
          + [pltpu.VMEM((CHUNK, HID), jnp.float32) for _ in range(NBUF)]
          + [pltpu.VMEM((CHUNK, HID), jnp.float32) for _ in range(NBUF)]
          + [pltpu.SemaphoreType.DMA for _ in range(2 * NBUF)]
      ),
      compiler_params=pltpu.CompilerParams(use_tc_tiling_on_sc=False),
  )
  return k(idx_flat, table)


def kernel(inputs, shared_weights):
  idx_flat = inputs.reshape(-1).astype(jnp.int32)
  return _run(idx_flat, shared_weights)


# speedup vs baseline: 1.0051x; 1.0051x over previous
import functools

import jax
import jax.numpy as jnp
from jax import lax
from jax.experimental import pallas as pl
from jax.experimental.pallas import tpu as pltpu
from jax.experimental.pallas import tpu_sc as plsc

VOCAB = 1000000
HID = 64
PAD = 0
SCALE = float(HID) ** 0.5

NC, NS, LANES = 2, 16, 16
NW = NC * NS
B_TOTAL = 4096 * 200
PER_W = B_TOTAL // NW
CHUNK = 320
NBUF = 2
N_CHUNKS = PER_W // CHUNK
assert N_CHUNKS % NBUF == 0


def _body(idx_hbm, table_hbm, out_hbm, idx_all,
          r0, r1, o0, o1, gs0, gs1, ss0, ss1):
  rbufs = [r0, r1]
  obufs = [o0, o1]
  gsems = [gs0, gs1]
  ssems = [ss0, ss1]

  wid = lax.axis_index("s") * NC + lax.axis_index("c")
  base = pl.multiple_of(wid * PER_W, 1024)

  pltpu.sync_copy(idx_hbm.at[pl.ds(base, PER_W)], idx_all)

  def start_gather(chunk, b):
    pltpu.async_copy(
        table_hbm.at[idx_all.at[pl.ds(chunk * CHUNK, CHUNK)]],
        rbufs[b], gsems[b])

  dnums = lax.GatherDimensionNumbers(
      offset_dims=(), collapsed_slice_dims=(0,), start_index_map=(0,))

  def splat(vec, j):
    return lax.gather(vec, jnp.full((LANES, 1), j, jnp.int32), dnums, (1,),
                      mode=lax.GatherScatterMode.PROMISE_IN_BOUNDS)

  def scale_chunk(chunk, b):
    rows = rbufs[b]
    dst = obufs[b]

    def group(g, carry):
      idxv = idx_all[pl.ds(chunk * CHUNK + g * LANES, LANES)]
      mv = jnp.minimum(idxv, 1).astype(jnp.float32) * SCALE
      for j in range(LANES):
        mspl = splat(mv, j)
        r = g * LANES + j
        for c in range(HID // LANES):
          dst[r, pl.ds(c * LANES, LANES)] = (
              rows[r, pl.ds(c * LANES, LANES)] * mspl)
      return carry

    lax.fori_loop(0, CHUNK // LANES, group, 0)

  for b in range(NBUF):
    start_gather(b, b)

  def outer(s, carry):
    for b in range(NBUF):
      chunk = s * NBUF + b
      pltpu.make_async_copy(
          table_hbm.at[idx_all.at[pl.ds(chunk * CHUNK, CHUNK)]],
          rbufs[b], gsems[b]).wait()

      @pl.when(chunk >= NBUF)
      def _():
        off0 = pl.multiple_of(base + (chunk - NBUF) * CHUNK, 64)
        pltpu.make_async_copy(
            obufs[b], out_hbm.at[pl.ds(off0, CHUNK)], ssems[b]).wait()

      scale_chunk(chunk, b)
      off = pl.multiple_of(base + chunk * CHUNK, 64)
      pltpu.async_copy(obufs[b], out_hbm.at[pl.ds(off, CHUNK)], ssems[b])

      @pl.when(chunk + NBUF < N_CHUNKS)
      def _():
        start_gather(chunk + NBUF, b)

    return carry

  lax.fori_loop(0, N_CHUNKS // NBUF, outer, 0)

  for b in range(NBUF):
    chunk = N_CHUNKS - NBUF + b
    off = pl.multiple_of(base + chunk * CHUNK, 64)
    pltpu.make_async_copy(
        obufs[b], out_hbm.at[pl.ds(off, CHUNK)], ssems[b]).wait()


@functools.partial(jax.jit, static_argnames=())
def _run(idx_flat, table):
  mesh = plsc.VectorSubcoreMesh(core_axis_name="c", subcore_axis_name="s")
  k = pl.kernel(
      _body,
      out_type=jax.ShapeDtypeStruct((B_TOTAL, HID), jnp.float32),
      mesh=mesh,
      scratch_types=(
          [pltpu.VMEM((PER_W,), jnp.int32)]
          + [pltpu.VMEM((CHUNK, HID), jnp.float32) for _ in range(NBUF)]
          + [pltpu.VMEM((CHUNK, HID), jnp.float32) for _ in range(NBUF)]
          + [pltpu.SemaphoreType.DMA for _ in range(2 * NBUF)]
      ),
      compiler_params=pltpu.CompilerParams(use_tc_tiling_on_sc=False),
  )
  return k(idx_flat, table)


def kernel(inputs, shared_weights):
  idx_flat = inputs.reshape(-1).astype(jnp.int32)
  out = _run(idx_flat, shared_weights)
  return out.reshape(inputs.shape + (HID,))
